# baseline (speedup 1.0000x reference)
import dataclasses
import functools

import jax
import jax.numpy as jnp
from jax import lax
from jax.experimental import pallas as pl
from jax.experimental.pallas import tpu as pltpu
from jax.experimental.pallas import tpu_sc as plsc

B, N, C, M = 8, 16384, 256, 2048
NC, NS = 2, 16
NW = NC * NS
TPB = NW // B
CPT = C // TPB
MPT = M // TPB
L = 16


def _compiler_params():
    cp = pltpu.CompilerParams()
    fields = pltpu.CompilerParams.__dataclass_fields__
    if "needs_layout_passes" in fields:
        cp = dataclasses.replace(cp, needs_layout_passes=False)
    if "use_tc_tiling_on_sc" in fields:
        cp = dataclasses.replace(cp, use_tc_tiling_on_sc=False)
    return cp


def _sc_gather(xyz_flat, features, inds):
    mesh = plsc.VectorSubcoreMesh(core_axis_name="c", subcore_axis_name="s")

    @functools.partial(
        pl.kernel,
        compiler_params=_compiler_params(),
        out_type=(
            jax.ShapeDtypeStruct((B, M * 3), jnp.float32),
            jax.ShapeDtypeStruct((B, C, M), jnp.float32),
        ),
        mesh=mesh,
        scratch_types=[
            pltpu.VMEM((M,), jnp.int32),
            pltpu.VMEM((N * 3,), jnp.float32),
            pltpu.VMEM((N,), jnp.float32),
            pltpu.VMEM((M,), jnp.float32),
            pltpu.VMEM((MPT * 3,), jnp.float32),
        ],
    )
    def k(xyz_hbm, feat_hbm, inds_hbm, oxyz_hbm, ofeat_hbm,
          inds_v, xyz_v, row_v, orow_v, oxyz_v):
        wid = lax.axis_index("c") * NS + lax.axis_index("s")
        b = wid // TPB
        q = wid % TPB

        pltpu.sync_copy(inds_hbm.at[b], inds_v)

        pltpu.sync_copy(xyz_hbm.at[b], xyz_v)
        mbase = q * MPT

        @pl.loop(0, MPT, step=L)
        def _(ml):
            idx = inds_v[pl.ds(mbase + ml, L)]
            mloc = ml + lax.iota(jnp.int32, L)
            for j in range(3):
                v = plsc.load_gather(xyz_v, [idx * 3 + j])
                plsc.store_scatter(oxyz_v, [mloc * 3 + j], v)

        pltpu.sync_copy(oxyz_v, oxyz_hbm.at[b, pl.ds(mbase * 3, MPT * 3)])

        cbase = q * CPT

        @pl.loop(0, CPT)
        def _(cl):
            c = cbase + cl
            pltpu.sync_copy(feat_hbm.at[b, c], row_v)

            @pl.loop(0, M, step=L)
            def _(ml):
                idx = inds_v[pl.ds(ml, L)]
                orow_v[pl.ds(ml, L)] = plsc.load_gather(row_v, [idx])

            pltpu.sync_copy(orow_v, ofeat_hbm.at[b, c])

    return k(xyz_flat, features, inds)


def kernel(xyz, features, sample_inds):
    inds32 = sample_inds.astype(jnp.int32)
    xyz_flat = xyz.reshape(B, N * 3)
    oxyz_flat, new_features = _sc_gather(xyz_flat, features, inds32)
    new_xyz = oxyz_flat.reshape(B, M, 3)
    return (new_xyz, new_features, sample_inds)


# speedup vs baseline: 1.3722x; 1.3722x over previous
import dataclasses
import functools

import jax
import jax.numpy as jnp
from jax import lax
from jax.experimental import pallas as pl
from jax.experimental.pallas import tpu as pltpu
from jax.experimental.pallas import tpu_sc as plsc

B, N, C, M = 8, 16384, 256, 2048
NC, NS = 2, 16
NW = NC * NS
TPB = NW // B
CPT = C // TPB
MPT = M // TPB
L = 16
G = 2
NG = CPT // G


def _compiler_params():
    cp = pltpu.CompilerParams()
    fields = pltpu.CompilerParams.__dataclass_fields__
    if "needs_layout_passes" in fields:
        cp = dataclasses.replace(cp, needs_layout_passes=False)
    if "use_tc_tiling_on_sc" in fields:
        cp = dataclasses.replace(cp, use_tc_tiling_on_sc=False)
    return cp


def _sc_gather(xyz_flat, feat, inds):
    mesh = plsc.VectorSubcoreMesh(core_axis_name="c", subcore_axis_name="s")

    @functools.partial(
        pl.kernel,
        compiler_params=_compiler_params(),
        out_type=(
            jax.ShapeDtypeStruct((B, M * 3), jnp.float32),
            jax.ShapeDtypeStruct((B, C, M), jnp.float32),
        ),
        mesh=mesh,
        scratch_types=[
            pltpu.VMEM((M,), jnp.int32),
            pltpu.VMEM((N * 3,), jnp.float32),
            pltpu.VMEM((G, N), jnp.float32),
            pltpu.VMEM((G, N), jnp.float32),
            pltpu.VMEM((G, M), jnp.float32),
            pltpu.VMEM((G, M), jnp.float32),
            pltpu.VMEM((MPT * 3,), jnp.float32),
            pltpu.SemaphoreType.DMA,
            pltpu.SemaphoreType.DMA,
            pltpu.SemaphoreType.DMA,
            pltpu.SemaphoreType.DMA,
        ],
    )
    def k(xyz_hbm, feat_hbm, inds_hbm, oxyz_hbm, ofeat_hbm,
          inds_v, xyz_v, buf_a, buf_b, obuf_a, obuf_b, oxyz_v,
          isem_a, isem_b, osem_a, osem_b):
        wid = lax.axis_index("c") * NS + lax.axis_index("s")
        b = wid // TPB
        q = wid % TPB
        cbase = q * CPT
        bufs = (buf_a, buf_b)
        obufs = (obuf_a, obuf_b)
        isems = (isem_a, isem_b)
        osems = (osem_a, osem_b)

        def src(gi):
            return feat_hbm.at[b, pl.ds(cbase + gi * G, G), :]

        def dst(gi):
            return ofeat_hbm.at[b, pl.ds(cbase + gi * G, G), :]

        pltpu.sync_copy(inds_hbm.at[b], inds_v)

        pltpu.async_copy(src(0), buf_a, isem_a)
        pltpu.async_copy(src(1), buf_b, isem_b)

        pltpu.sync_copy(xyz_hbm.at[b], xyz_v)
        mbase = q * MPT

        @pl.loop(0, MPT, step=L)
        def _(ml):
            idx = inds_v[pl.ds(mbase + ml, L)]
            mloc = ml + lax.iota(jnp.int32, L)
            for j in range(3):
                v = plsc.load_gather(xyz_v, [idx * 3 + j])
                plsc.store_scatter(oxyz_v, [mloc * 3 + j], v)

        pltpu.sync_copy(oxyz_v, oxyz_hbm.at[b, pl.ds(mbase * 3, MPT * 3)])

        @pl.loop(0, NG, step=2)
        def _(g):
            for kb in range(2):
                buf, obuf = bufs[kb], obufs[kb]
                isem, osem = isems[kb], osems[kb]
                gi = g + kb
                pltpu.make_async_copy(src(gi), buf, isem).wait()

                @pl.when(gi >= 2)
                def _():
                    pltpu.make_async_copy(obuf, dst(gi - 2), osem).wait()

                @pl.loop(0, M, step=L)
                def _(m):
                    idx = inds_v[pl.ds(m, L)]
                    for r in range(G):
                        rv = jnp.full((L,), r, jnp.int32)
                        obuf[r, pl.ds(m, L)] = plsc.load_gather(buf, [rv, idx])

                pltpu.async_copy(obuf, dst(gi), osem)

                @pl.when(gi + 2 < NG)
                def _():
                    pltpu.async_copy(src(gi + 2), buf, isem)

        pltpu.make_async_copy(obuf_a, dst(NG - 2), osem_a).wait()
        pltpu.make_async_copy(obuf_b, dst(NG - 1), osem_b).wait()

    return k(xyz_flat, feat, inds)


def kernel(xyz, features, sample_inds):
    inds32 = sample_inds.astype(jnp.int32)
    xyz_flat = xyz.reshape(B, N * 3)
    oxyz_flat, new_features = _sc_gather(xyz_flat, features, inds32)
    new_xyz = oxyz_flat.reshape(B, M, 3)
    return (new_xyz, new_features, sample_inds)
